# Initial kernel scaffold; baseline (speedup 1.0000x reference)
import jax
import jax.numpy as jnp
from jax.experimental import pallas as pl


def kernel(input_nlp_embedding, input_r, in_elapsed_time, output_nlp_embedding, response_table, et_W, et_b, position_table):
    raise NotImplementedError("write your pallas kernel here")


# speedup vs baseline: 4.5749x; 4.5749x over previous
import jax
import jax.numpy as jnp
from jax.experimental import pallas as pl

B = 1024
S = 200
D = 128
BB = 16


def _body(inp_ref, r_ref, et_ref, rt_ref, w_ref, b_ref, pos_ref, enc_ref, dec_ref):
    pos = pos_ref[...]
    enc_ref[...] = inp_ref[...] + pos[None]
    rr = r_ref[...][:, :, None]
    e = et_ref[...][:, :, None]
    rt = rt_ref[...]
    resp = jnp.where(
        rr == 0, rt[0],
        jnp.where(rr == 1, rt[1], jnp.where(rr == 2, rt[2], rt[3])))
    dec_ref[...] = resp + e * w_ref[...] + b_ref[...] + pos[None]


def kernel(input_nlp_embedding, input_r, in_elapsed_time, output_nlp_embedding,
           response_table, et_W, et_b, position_table):
    et2 = in_elapsed_time.reshape(B, S)
    b2 = et_b.reshape(1, D)
    grid = (B // BB,)
    enc, dec = pl.pallas_call(
        _body,
        grid=grid,
        in_specs=[
            pl.BlockSpec((BB, S, D), lambda i: (i, 0, 0)),
            pl.BlockSpec((BB, S), lambda i: (i, 0)),
            pl.BlockSpec((BB, S), lambda i: (i, 0)),
            pl.BlockSpec((4, D), lambda i: (0, 0)),
            pl.BlockSpec((1, D), lambda i: (0, 0)),
            pl.BlockSpec((1, D), lambda i: (0, 0)),
            pl.BlockSpec((S, D), lambda i: (0, 0)),
        ],
        out_specs=[
            pl.BlockSpec((BB, S, D), lambda i: (i, 0, 0)),
            pl.BlockSpec((BB, S, D), lambda i: (i, 0, 0)),
        ],
        out_shape=[
            jax.ShapeDtypeStruct((B, S, D), jnp.float32),
            jax.ShapeDtypeStruct((B, S, D), jnp.float32),
        ],
    )(input_nlp_embedding, input_r, et2, response_table, et_W, b2,
      position_table)
    return (enc, dec, output_nlp_embedding)
